# baseline (speedup 1.0000x reference)
import jax
import jax.numpy as jnp
from jax.experimental import pallas as pl
from jax.experimental.pallas import tpu as pltpu

BM = 256
N = 4096
NB = N // BM


def _dot(a, b):
    return jnp.dot(a, b, preferred_element_type=jnp.float32)


def _bdot(a, b):
    return jnp.dot(a.astype(jnp.bfloat16), b.astype(jnp.bfloat16),
                   preferred_element_type=jnp.float32)


def _body(u_adj_a, u_adj_b, inner_a, inner_b, v_adj_a, v_adj_b, u_attr, v_attr,
          Wn_v, Ws_v, b_v, Wn_u, Ws_u, b_u, W_in1, b_in1,
          Wn_v2, Ws_v2, b_v2, Wn_u2, Ws_u2, b_u2, W_in2, b_in2,
          u_emb2, v_emb2, rating,
          P_u, S_u, P_v, S_v, Q1, Pv2, Su2, Pu2, Q2, VT):
    i = pl.program_id(0)
    p = i // NB
    r = i % NB
    HB = BM // 2
    rows = (pl.ds(r * BM, HB), pl.ds(r * BM + HB, HB))

    @pl.when(i == 0)
    def _():
        P_u[...] = _dot(v_attr[...], Wn_u[...])
        S_u[...] = _dot(u_attr[...], Ws_u[...]) + b_u[...]
        P_v[...] = _dot(u_attr[...], Wn_v[...])
        S_v[...] = _dot(v_attr[...], Ws_v[...]) + b_v[...]

    @pl.when(p == 0)
    def _():
        for half, row in zip((u_adj_a, u_adj_b), rows):
            u1 = jnp.maximum(_bdot(half[...], P_u[...]) + S_u[row, :], 0.0)
            Q1[row, :] = _dot(u1, W_in1[...])

    @pl.when(p == 1)
    def _():
        for half, row in zip((inner_a, inner_b), rows):
            t = jnp.maximum(_bdot(half[...], Q1[...]) + b_in1[...], 0.0)
            Pv2[row, :] = _dot(t, Wn_v2[...])
            Su2[row, :] = _dot(t, Ws_u2[...]) + b_u2[...]

    @pl.when(p == 2)
    def _():
        for k, (half, row) in enumerate(zip((v_adj_a, v_adj_b), rows)):
            a = half[...]
            ve = jnp.maximum(_bdot(a, P_v[...]) + S_v[row, :], 0.0)
            ve2 = jnp.maximum(
                _bdot(a, Pv2[...]) + _dot(ve, Ws_v2[...]) + b_v2[...], 0.0)
            v_emb2[pl.ds(k * HB, HB), :] = ve2
            VT[:, row] = ve2.T
            Pu2[row, :] = _dot(ve, Wn_u2[...])

    @pl.when(p == 3)
    def _():
        for half, row in zip((u_adj_a, u_adj_b), rows):
            u2a = jnp.maximum(_bdot(half[...], Pu2[...]) + Su2[row, :], 0.0)
            Q2[row, :] = _dot(u2a, W_in2[...])

    @pl.when(p == 4)
    def _():
        for k, (half, row) in enumerate(zip((inner_a, inner_b), rows)):
            t = jnp.maximum(_bdot(half[...], Q2[...]) + b_in2[...], 0.0)
            u_emb2[pl.ds(k * HB, HB), :] = t
            rating[pl.ds(k * HB, HB), :] = jnp.tanh(_bdot(t, VT[...]))


def _im_uadj(h):
    def im(i):
        r = i % NB
        blk = jnp.where(i < NB, r, jnp.where(i < 3 * NB, 0,
              jnp.where(i < 4 * NB, r, NB - 1)))
        return (2 * blk + h, 0)
    return im


def _im_inner(h):
    def im(i):
        r = i % NB
        blk = jnp.where(i < NB, 0, jnp.where(i < 2 * NB, r,
              jnp.where(i < 4 * NB, 0, r)))
        return (2 * blk + h, 0)
    return im


def _im_vadj(h):
    def im(i):
        r = i % NB
        blk = jnp.where(i < 2 * NB, 0, jnp.where(i < 3 * NB, r, NB - 1))
        return (2 * blk + h, 0)
    return im


def _im_p2out(i):
    r = i % NB
    return (jnp.where(i < 2 * NB, 0, jnp.where(i < 3 * NB, r, NB - 1)), 0)


def _im_p4out(i):
    r = i % NB
    return (jnp.where(i < 4 * NB, 0, r), 0)


def _full(shape):
    return pl.BlockSpec(shape, lambda i: (0,) * len(shape))


@jax.jit
def kernel(u_attr, v_attr, u_adj, v_adj, u_adj_inner,
           Wn_v, Ws_v, b_v, Wn_u, Ws_u, b_u, W_in1, b_in1,
           Wn_v2, Ws_v2, b_v2, Wn_u2, Ws_u2, b_u2, W_in2, b_in2):
    f32 = jnp.float32
    H = Wn_v.shape[1]
    O = Wn_v2.shape[1]
    DU = u_attr.shape[1]
    DV = v_attr.shape[1]
    b_v = b_v.reshape(1, H)
    b_u = b_u.reshape(1, H)
    b_in1 = b_in1.reshape(1, H)
    b_v2 = b_v2.reshape(1, O)
    b_u2 = b_u2.reshape(1, O)
    b_in2 = b_in2.reshape(1, O)

    u_emb2, v_emb2, rating = pl.pallas_call(
        _body,
        grid=(5 * NB,),
        in_specs=[
            pl.BlockSpec((BM // 2, N), _im_uadj(0)),
            pl.BlockSpec((BM // 2, N), _im_uadj(1)),
            pl.BlockSpec((BM // 2, N), _im_inner(0)),
            pl.BlockSpec((BM // 2, N), _im_inner(1)),
            pl.BlockSpec((BM // 2, N), _im_vadj(0)),
            pl.BlockSpec((BM // 2, N), _im_vadj(1)),
            _full((N, DU)), _full((N, DV)),
            _full((DU, H)), _full((DV, H)), _full((1, H)),
            _full((DV, H)), _full((DU, H)), _full((1, H)),
            _full((H, H)), _full((1, H)),
            _full((H, O)), _full((H, O)), _full((1, O)),
            _full((H, O)), _full((H, O)), _full((1, O)),
            _full((O, O)), _full((1, O)),
        ],
        out_specs=[
            pl.BlockSpec((BM, O), _im_p4out),
            pl.BlockSpec((BM, O), _im_p2out),
            pl.BlockSpec((BM, N), _im_p4out),
        ],
        out_shape=[
            jax.ShapeDtypeStruct((N, O), f32),
            jax.ShapeDtypeStruct((N, O), f32),
            jax.ShapeDtypeStruct((N, N), f32),
        ],
        scratch_shapes=[
            pltpu.VMEM((N, H), f32), pltpu.VMEM((N, H), f32),
            pltpu.VMEM((N, H), f32), pltpu.VMEM((N, H), f32),
            pltpu.VMEM((N, H), f32),
            pltpu.VMEM((N, O), f32), pltpu.VMEM((N, O), f32),
            pltpu.VMEM((N, O), f32), pltpu.VMEM((N, O), f32),
            pltpu.VMEM((O, N), f32),
        ],
    )(u_adj, u_adj, u_adj_inner, u_adj_inner, v_adj, v_adj, u_attr, v_attr,
      Wn_v, Ws_v, b_v, Wn_u, Ws_u, b_u, W_in1, b_in1,
      Wn_v2, Ws_v2, b_v2, Wn_u2, Ws_u2, b_u2, W_in2, b_in2)

    return (u_emb2, v_emb2, rating)


# speedup vs baseline: 1.1352x; 1.1352x over previous
import jax
import jax.numpy as jnp
from jax.experimental import pallas as pl
from jax.experimental.pallas import tpu as pltpu

BM = 256
N = 4096
NB = N // BM


def _dot(a, b):
    return jnp.dot(a, b, preferred_element_type=jnp.float32)


def _b16(x):
    return x.astype(jnp.bfloat16)


def _body(u_adj, inner, v_adj, u_attr, v_attr,
          Wn_v, Ws_v, b_v, Wn_u, Ws_u, b_u, W_in1, b_in1,
          Wn_v2, Ws_v2, b_v2, Wn_u2, Ws_u2, b_u2, W_in2, b_in2,
          u_emb2, v_emb2, rating,
          PSu, PSv, MID, VT, IC, DB, sem):
    i = pl.program_id(0)
    p = i // NB
    r = i % NB
    row = pl.ds(r * BM, BM)
    slot = jax.lax.rem(i, 2)

    def issue(j):
        q = j // NB
        rows = pl.ds((j % NB) * BM, BM)
        dst = DB.at[jax.lax.rem(j, 2)]
        s = sem.at[jax.lax.rem(j, 2)]

        @pl.when(jnp.logical_or(q == 0, q == 3))
        def _():
            pltpu.make_async_copy(u_adj.at[rows, :], dst, s).start()

        @pl.when(q == 1)
        def _():
            pltpu.make_async_copy(inner.at[rows, :], dst, s).start()

        @pl.when(q == 2)
        def _():
            pltpu.make_async_copy(v_adj.at[rows, :], dst, s).start()

    @pl.when(i == 0)
    def _():
        issue(0)
        PSu[:, 0:64] = _dot(v_attr[...], Wn_u[...])
        PSu[:, 64:128] = _dot(u_attr[...], Ws_u[...]) + b_u[...]
        PSv[:, 0:64] = _dot(u_attr[...], Wn_v[...])
        PSv[:, 64:128] = _dot(v_attr[...], Ws_v[...]) + b_v[...]

    @pl.when(i + 1 < 4 * NB)
    def _():
        issue(i + 1)

    @pl.when(i < 4 * NB)
    def _():
        pltpu.make_async_copy(DB.at[slot], DB.at[slot], sem.at[slot]).wait()

    @pl.when(p == 0)
    def _():
        a = _b16(DB[slot])
        u1 = jnp.maximum(
            _dot(a, _b16(PSu[:, 0:64])) + PSu[row, 64:128], 0.0)
        MID[row, 0:64] = _dot(u1, W_in1[...])

    @pl.when(p == 1)
    def _():
        a = _b16(DB[slot])
        IC[row, :] = a
        t = jnp.maximum(_dot(a, _b16(MID[:, 0:64])) + b_in1[...], 0.0)
        MID[row, 64:72] = _dot(t, Wn_v2[...])
        MID[row, 72:80] = _dot(t, Ws_u2[...]) + b_u2[...]

    @pl.when(p == 2)
    def _():
        a = _b16(DB[slot])
        ve = jnp.maximum(
            _dot(a, _b16(PSv[:, 0:64])) + PSv[row, 64:128], 0.0)
        ve2 = jnp.maximum(
            _dot(a, _b16(MID[:, 64:72])) + _dot(ve, Ws_v2[...])
            + b_v2[...], 0.0)
        v_emb2[...] = ve2
        VT[:, row] = ve2.T
        MID[row, 80:88] = _dot(ve, Wn_u2[...])

    @pl.when(p == 3)
    def _():
        a = _b16(DB[slot])
        u2a = jnp.maximum(
            _dot(a, _b16(MID[:, 80:88])) + MID[row, 72:80], 0.0)
        MID[row, 88:96] = _dot(u2a, W_in2[...])

    @pl.when(p == 4)
    def _():
        t = jnp.maximum(
            _dot(IC[row, :], _b16(MID[:, 88:96])) + b_in2[...], 0.0)
        u_emb2[...] = t
        rating[...] = jnp.tanh(_dot(_b16(t), _b16(VT[...])))


def _im_p2out(i):
    r = i % NB
    return (jnp.where(i < 2 * NB, 0, jnp.where(i < 3 * NB, r, NB - 1)), 0)


def _im_p4out(i):
    r = i % NB
    return (jnp.where(i < 4 * NB, 0, r), 0)


def _full(shape):
    return pl.BlockSpec(shape, lambda i: (0,) * len(shape))


_ANY = pl.BlockSpec(memory_space=pl.ANY)


@jax.jit
def kernel(u_attr, v_attr, u_adj, v_adj, u_adj_inner,
           Wn_v, Ws_v, b_v, Wn_u, Ws_u, b_u, W_in1, b_in1,
           Wn_v2, Ws_v2, b_v2, Wn_u2, Ws_u2, b_u2, W_in2, b_in2):
    f32 = jnp.float32
    bf16 = jnp.bfloat16
    H = Wn_v.shape[1]
    O = Wn_v2.shape[1]
    DU = u_attr.shape[1]
    DV = v_attr.shape[1]
    b_v = b_v.reshape(1, H)
    b_u = b_u.reshape(1, H)
    b_in1 = b_in1.reshape(1, H)
    b_v2 = b_v2.reshape(1, O)
    b_u2 = b_u2.reshape(1, O)
    b_in2 = b_in2.reshape(1, O)

    u_emb2, v_emb2, rating = pl.pallas_call(
        _body,
        grid=(5 * NB,),
        in_specs=[
            _ANY, _ANY, _ANY,
            _full((N, DU)), _full((N, DV)),
            _full((DU, H)), _full((DV, H)), _full((1, H)),
            _full((DV, H)), _full((DU, H)), _full((1, H)),
            _full((H, H)), _full((1, H)),
            _full((H, O)), _full((H, O)), _full((1, O)),
            _full((H, O)), _full((H, O)), _full((1, O)),
            _full((O, O)), _full((1, O)),
        ],
        out_specs=[
            pl.BlockSpec((BM, O), _im_p4out),
            pl.BlockSpec((BM, O), _im_p2out),
            pl.BlockSpec((BM, N), _im_p4out),
        ],
        out_shape=[
            jax.ShapeDtypeStruct((N, O), f32),
            jax.ShapeDtypeStruct((N, O), f32),
            jax.ShapeDtypeStruct((N, N), f32),
        ],
        scratch_shapes=[
            pltpu.VMEM((N, 128), f32),
            pltpu.VMEM((N, 128), f32),
            pltpu.VMEM((N, 128), f32),
            pltpu.VMEM((O, N), f32),
            pltpu.VMEM((N, N), bf16),
            pltpu.VMEM((2, BM, N), f32),
            pltpu.SemaphoreType.DMA((2,)),
        ],
        compiler_params=pltpu.CompilerParams(
            vmem_limit_bytes=64 * 1024 * 1024),
    )(u_adj, u_adj_inner, v_adj, u_attr, v_attr,
      Wn_v, Ws_v, b_v, Wn_u, Ws_u, b_u, W_in1, b_in1,
      Wn_v2, Ws_v2, b_v2, Wn_u2, Ws_u2, b_u2, W_in2, b_in2)

    return (u_emb2, v_emb2, rating)


# speedup vs baseline: 1.3718x; 1.2084x over previous
import jax
import jax.numpy as jnp
from jax.experimental import pallas as pl
from jax.experimental.pallas import tpu as pltpu

BM = 256
N = 4096
NB = N // BM


def _dot(a, b):
    return jnp.dot(a, b, preferred_element_type=jnp.float32)


def _b16(x):
    return x.astype(jnp.bfloat16)


def _body(u_adj, inner, v_adj, u_attr, v_attr,
          Wn_v, Ws_v, b_v, Wn_u, Ws_u, b_u, W_in1, b_in1,
          Wn_v2, Ws_v2, b_v2, Wn_u2, Ws_u2, b_u2, W_in2, b_in2,
          u_emb2, v_emb2, rating,
          PSu, PSv, MID, VT, IC, DB, sem):
    i = pl.program_id(0)
    p = i // NB
    r = i % NB
    row = pl.ds(r * BM, BM)
    slot = jax.lax.rem(i, 3)

    def issue(j):
        q = j // NB
        rows = pl.ds((j % NB) * BM, BM)
        dst = DB.at[jax.lax.rem(j, 3)]
        s = sem.at[jax.lax.rem(j, 3)]

        @pl.when(jnp.logical_or(q == 0, q == 3))
        def _():
            pltpu.make_async_copy(u_adj.at[rows, :], dst, s).start()

        @pl.when(q == 1)
        def _():
            pltpu.make_async_copy(inner.at[rows, :], dst, s).start()

        @pl.when(q == 2)
        def _():
            pltpu.make_async_copy(v_adj.at[rows, :], dst, s).start()

    @pl.when(i == 0)
    def _():
        issue(0)
        issue(1)
        issue(2)
        PSu[:, 0:64] = _dot(v_attr[...], Wn_u[...])
        PSv[:, 64:128] = _dot(v_attr[...], Ws_v[...]) + b_v[...]

    @pl.when(jnp.logical_and(i + 2 < 4 * NB, i > 0))
    def _():
        issue(i + 2)

    @pl.when(i < 4 * NB)
    def _():
        pltpu.make_async_copy(DB.at[slot], DB.at[slot], sem.at[slot]).wait()

    @pl.when(p == 0)
    def _():
        su = _dot(u_attr[...], Ws_u[...]) + b_u[...]
        PSv[row, 0:64] = _dot(u_attr[...], Wn_v[...])
        a = _b16(DB[slot])
        u1 = jnp.maximum(_dot(a, _b16(PSu[:, 0:64])) + su, 0.0)
        MID[row, 0:64] = _dot(u1, W_in1[...])

    @pl.when(p == 1)
    def _():
        a = _b16(DB[slot])
        IC[row, :] = a
        t = jnp.maximum(_dot(a, _b16(MID[:, 0:64])) + b_in1[...], 0.0)
        MID[row, 64:72] = _dot(t, Wn_v2[...])
        MID[row, 72:80] = _dot(t, Ws_u2[...]) + b_u2[...]

    @pl.when(p == 2)
    def _():
        a = _b16(DB[slot])
        ve = jnp.maximum(
            _dot(a, _b16(PSv[:, 0:64])) + PSv[row, 64:128], 0.0)
        ve2 = jnp.maximum(
            _dot(a, _b16(MID[:, 64:72])) + _dot(ve, Ws_v2[...])
            + b_v2[...], 0.0)
        v_emb2[...] = ve2
        VT[:, row] = ve2.T
        MID[row, 80:88] = _dot(ve, Wn_u2[...])

    @pl.when(p == 3)
    def _():
        a = _b16(DB[slot])
        u2a = jnp.maximum(
            _dot(a, _b16(MID[:, 80:88])) + MID[row, 72:80], 0.0)
        MID[row, 88:96] = _dot(u2a, W_in2[...])

    @pl.when(p == 4)
    def _():
        t = jnp.maximum(
            _dot(IC[row, :], _b16(MID[:, 88:96])) + b_in2[...], 0.0)
        u_emb2[...] = t
        rating[...] = jnp.tanh(_dot(_b16(t), _b16(VT[...])))


def _im_p2out(i):
    r = i % NB
    return (jnp.where(i < 2 * NB, 0, jnp.where(i < 3 * NB, r, NB - 1)), 0)


def _im_p4out(i):
    r = i % NB
    return (jnp.where(i < 4 * NB, 0, r), 0)


def _full(shape):
    return pl.BlockSpec(shape, lambda i: (0,) * len(shape))


_ANY = pl.BlockSpec(memory_space=pl.ANY)


@jax.jit
def kernel(u_attr, v_attr, u_adj, v_adj, u_adj_inner,
           Wn_v, Ws_v, b_v, Wn_u, Ws_u, b_u, W_in1, b_in1,
           Wn_v2, Ws_v2, b_v2, Wn_u2, Ws_u2, b_u2, W_in2, b_in2):
    f32 = jnp.float32
    bf16 = jnp.bfloat16
    H = Wn_v.shape[1]
    O = Wn_v2.shape[1]
    DU = u_attr.shape[1]
    DV = v_attr.shape[1]
    b_v = b_v.reshape(1, H)
    b_u = b_u.reshape(1, H)
    b_in1 = b_in1.reshape(1, H)
    b_v2 = b_v2.reshape(1, O)
    b_u2 = b_u2.reshape(1, O)
    b_in2 = b_in2.reshape(1, O)

    u_emb2, v_emb2, rating = pl.pallas_call(
        _body,
        grid=(5 * NB,),
        in_specs=[
            _ANY, _ANY, _ANY,
            pl.BlockSpec((BM, DU),
                         lambda i: (jnp.where(i < NB, i % NB, NB - 1), 0)),
            _full((N, DV)),
            _full((DU, H)), _full((DV, H)), _full((1, H)),
            _full((DV, H)), _full((DU, H)), _full((1, H)),
            _full((H, H)), _full((1, H)),
            _full((H, O)), _full((H, O)), _full((1, O)),
            _full((H, O)), _full((H, O)), _full((1, O)),
            _full((O, O)), _full((1, O)),
        ],
        out_specs=[
            pl.BlockSpec((BM, O), _im_p4out),
            pl.BlockSpec((BM, O), _im_p2out),
            pl.BlockSpec((BM, N), _im_p4out),
        ],
        out_shape=[
            jax.ShapeDtypeStruct((N, O), f32),
            jax.ShapeDtypeStruct((N, O), f32),
            jax.ShapeDtypeStruct((N, N), f32),
        ],
        scratch_shapes=[
            pltpu.VMEM((N, 128), f32),
            pltpu.VMEM((N, 128), f32),
            pltpu.VMEM((N, 128), f32),
            pltpu.VMEM((O, N), f32),
            pltpu.VMEM((N, N), bf16),
            pltpu.VMEM((3, BM, N), f32),
            pltpu.SemaphoreType.DMA((3,)),
        ],
        compiler_params=pltpu.CompilerParams(
            vmem_limit_bytes=64 * 1024 * 1024),
    )(u_adj, u_adj_inner, v_adj, u_attr, v_attr,
      Wn_v, Ws_v, b_v, Wn_u, Ws_u, b_u, W_in1, b_in1,
      Wn_v2, Ws_v2, b_v2, Wn_u2, Ws_u2, b_u2, W_in2, b_in2)

    return (u_emb2, v_emb2, rating)
